# baseline (speedup 1.0000x reference)
import functools

import jax
import jax.numpy as jnp
from jax import lax
from jax.experimental import pallas as pl
from jax.experimental.pallas import tpu as pltpu
from jax.experimental.pallas import tpu_sc as plsc

_LANES = 16


@functools.lru_cache(maxsize=None)
def _build(B, C, K):
    NC, NS = 2, 16
    NW = NC * NS
    RW = B // NW
    OUTW = _LANES

    RI = 16
    PASS_ROWS = 32
    NPASS = RW // PASS_ROWS
    NQ = PASS_ROWS // RI
    NF = C // _LANES
    REM = C - NF * _LANES

    mesh = plsc.VectorSubcoreMesh(core_axis_name="c", subcore_axis_name="s")

    @functools.partial(
        pl.kernel,
        mesh=mesh,
        out_type=jax.ShapeDtypeStruct((B, OUTW), jnp.float32),
        compiler_params=pltpu.CompilerParams(needs_layout_passes=False),
        scratch_types=[
            pltpu.VMEM((PASS_ROWS, C), jnp.float32),
            pltpu.VMEM((PASS_ROWS, C), jnp.float32),
            pltpu.VMEM((RW,), jnp.int32),
            pltpu.VMEM((RW, OUTW), jnp.float32),
            pltpu.SemaphoreType.DMA,
            pltpu.SemaphoreType.DMA,
        ],
    )
    def sc_mine(logits_hbm, targets_hbm, out_hbm, bufa, bufb, tgt_v, out_v,
                sema, semb):
        wid = lax.axis_index("s") * NC + lax.axis_index("c")
        row0 = wid * RW
        pltpu.sync_copy(targets_hbm.at[pl.ds(row0, RW)], tgt_v)

        iota = lax.iota(jnp.int32, _LANES)
        ninf = jnp.full((_LANES,), -jnp.inf, jnp.float32)

        bufs = [bufa, bufb]
        sems = [sema, semb]

        def merge(t, v):
            vd, _ = plsc.sort_key_val(v, v, descending=True)
            m = jnp.maximum(t, vd)
            ts, _ = plsc.sort_key_val(m, m)
            return ts

        def start(p):
            return pltpu.async_copy(
                logits_hbm.at[pl.ds(row0 + p * PASS_ROWS, PASS_ROWS), :],
                bufs[p % 2],
                sems[p % 2],
            )

        handles = [start(0), None]
        for p in range(NPASS):
            if p + 1 < NPASS:
                handles[(p + 1) % 2] = start(p + 1)
            handles[p % 2].wait()
            buf = bufs[p % 2]

            def quad_body(q, _, p=p, buf=buf):
                rb = q * RI
                tspl = [
                    plsc.load_gather(
                        tgt_v,
                        [jnp.full((_LANES,), p * PASS_ROWS + rb + i, jnp.int32)],
                    )
                    for i in range(RI)
                ]

                def jbody(j, ts):
                    cols = jnp.full((_LANES,), j * _LANES, jnp.int32) + iota
                    out = []
                    for i in range(RI):
                        v = buf[rb + i, pl.ds(j * _LANES, _LANES)]
                        v = jnp.where(cols == tspl[i], ninf, v)
                        out.append(merge(ts[i], v))
                    return tuple(out)

                ts = lax.fori_loop(0, NF, jbody, (ninf,) * RI)

                if REM:
                    colst = jnp.full((_LANES,), C - _LANES, jnp.int32) + iota
                    tail = []
                    for i in range(RI):
                        v = buf[rb + i, pl.ds(C - _LANES, _LANES)]
                        v = jnp.where(iota < _LANES - REM, ninf, v)
                        v = jnp.where(colst == tspl[i], ninf, v)
                        tail.append(merge(ts[i], v))
                    ts = tuple(tail)

                for i in range(RI):
                    rev = lax.rev(ts[i], (0,))
                    posv = plsc.load_gather(
                        buf, [jnp.full((_LANES,), rb + i, jnp.int32), tspl[i]]
                    )
                    row_vec = jnp.where(
                        iota == 0, posv, jnp.where(iota < K, rev, ninf)
                    )
                    out_v[p * PASS_ROWS + rb + i, :] = row_vec
                return 0

            lax.fori_loop(0, NQ, quad_body, 0)

        pltpu.sync_copy(out_v, out_hbm.at[pl.ds(row0, RW), :])

    def tc_finish(res_ref, out_ref):
        x = res_ref[...] * 10.0
        m = jnp.max(x, axis=1, keepdims=True)
        s = jnp.sum(jnp.exp(x - m), axis=1)
        lse = m[:, 0] + jnp.log(s)
        loss = lse - x[:, 0]
        out_ref[...] = (jnp.sum(loss) * (1.0 / B)).reshape(1, 1)

    tc_call = pl.pallas_call(
        tc_finish,
        out_shape=jax.ShapeDtypeStruct((1, 1), jnp.float32),
    )

    return sc_mine, tc_call


def kernel(logits, targets):
    B, C = logits.shape
    K = int(0.01 * (C - 1))
    targets = targets.astype(jnp.int32)
    h = B // 2
    sc_half, _ = _build(h, C, K)
    _, fin = _build(B, C, K)
    res = jnp.concatenate(
        [sc_half(logits[:h], targets[:h]), sc_half(logits[h:], targets[h:])]
    )
    return fin(res)[0, 0]


# speedup vs baseline: 1.3767x; 1.3767x over previous
import functools

import jax
import jax.numpy as jnp
from jax import lax
from jax.experimental import pallas as pl
from jax.experimental.pallas import tpu as pltpu
from jax.experimental.pallas import tpu_sc as plsc

_LANES = 16


@functools.lru_cache(maxsize=None)
def _build(B, C, K):
    NC, NS = 2, 16
    NW = NC * NS
    RW = B // NW
    OUTW = _LANES

    RI = 16
    PASS_ROWS = 32
    NPASS = RW // PASS_ROWS
    NQ = PASS_ROWS // RI
    NF = C // _LANES
    REM = C - NF * _LANES

    mesh = plsc.VectorSubcoreMesh(core_axis_name="c", subcore_axis_name="s")

    @functools.partial(
        pl.kernel,
        mesh=mesh,
        out_type=jax.ShapeDtypeStruct((B, OUTW), jnp.float32),
        compiler_params=pltpu.CompilerParams(needs_layout_passes=False),
        scratch_types=[
            pltpu.VMEM((PASS_ROWS, C), jnp.float32),
            pltpu.VMEM((PASS_ROWS, C), jnp.float32),
            pltpu.VMEM((RW,), jnp.int32),
            pltpu.VMEM((RW, OUTW), jnp.float32),
            pltpu.SemaphoreType.DMA,
            pltpu.SemaphoreType.DMA,
        ],
    )
    def sc_mine(logits_hbm, targets_hbm, out_hbm, bufa, bufb, tgt_v, out_v,
                sema, semb):
        wid = lax.axis_index("s") * NC + lax.axis_index("c")
        row0 = wid * RW
        pltpu.sync_copy(targets_hbm.at[pl.ds(row0, RW)], tgt_v)

        iota = lax.iota(jnp.int32, _LANES)
        ninf = jnp.full((_LANES,), -jnp.inf, jnp.float32)

        bufs = [bufa, bufb]
        sems = [sema, semb]

        def merge(t, v):
            vd, _ = plsc.sort_key_val(v, v, descending=True)
            m = jnp.maximum(t, vd)
            ts, _ = plsc.sort_key_val(m, m)
            return ts

        def start(p):
            return pltpu.async_copy(
                logits_hbm.at[pl.ds(row0 + p * PASS_ROWS, PASS_ROWS), :],
                bufs[p % 2],
                sems[p % 2],
            )

        handles = [start(0), None]
        for p in range(NPASS):
            if p + 1 < NPASS:
                handles[(p + 1) % 2] = start(p + 1)
            handles[p % 2].wait()
            buf = bufs[p % 2]

            def quad_body(q, _, p=p, buf=buf):
                rb = q * RI
                tspl = [
                    plsc.load_gather(
                        tgt_v,
                        [jnp.full((_LANES,), p * PASS_ROWS + rb + i, jnp.int32)],
                    )
                    for i in range(RI)
                ]

                def jbody(j, ts):
                    cols = jnp.full((_LANES,), j * _LANES, jnp.int32) + iota
                    out = []
                    for i in range(RI):
                        v = buf[rb + i, pl.ds(j * _LANES, _LANES)]
                        v = jnp.where(cols == tspl[i], ninf, v)
                        out.append(merge(ts[i], v))
                    return tuple(out)

                ts = lax.fori_loop(0, NF, jbody, (ninf,) * RI)

                if REM:
                    colst = jnp.full((_LANES,), C - _LANES, jnp.int32) + iota
                    tail = []
                    for i in range(RI):
                        v = buf[rb + i, pl.ds(C - _LANES, _LANES)]
                        v = jnp.where(iota < _LANES - REM, ninf, v)
                        v = jnp.where(colst == tspl[i], ninf, v)
                        tail.append(merge(ts[i], v))
                    ts = tuple(tail)

                for i in range(RI):
                    rev = lax.rev(ts[i], (0,))
                    posv = plsc.load_gather(
                        buf, [jnp.full((_LANES,), rb + i, jnp.int32), tspl[i]]
                    )
                    row_vec = jnp.where(
                        iota == 0, posv, jnp.where(iota < K, rev, ninf)
                    )
                    out_v[p * PASS_ROWS + rb + i, :] = row_vec
                return 0

            lax.fori_loop(0, NQ, quad_body, 0)

        pltpu.sync_copy(out_v, out_hbm.at[pl.ds(row0, RW), :])

    def tc_finish(res_ref, out_ref):
        x = res_ref[...] * 10.0
        m = jnp.max(x, axis=1, keepdims=True)
        s = jnp.sum(jnp.exp(x - m), axis=1)
        lse = m[:, 0] + jnp.log(s)
        loss = lse - x[:, 0]
        out_ref[...] = (jnp.sum(loss) * (1.0 / B)).reshape(1, 1)

    tc_call = pl.pallas_call(
        tc_finish,
        out_shape=jax.ShapeDtypeStruct((1, 1), jnp.float32),
    )

    def run(logits, targets):
        res = sc_mine(logits, targets)
        return tc_call(res)[0, 0]

    return run


def kernel(logits, targets):
    B, C = logits.shape
    K = int(0.01 * (C - 1))
    return _build(B, C, K)(logits, targets.astype(jnp.int32))
